# baseline (speedup 1.0000x reference)
import functools

import jax
import jax.numpy as jnp
from jax import lax
from jax.experimental import pallas as pl
from jax.experimental.pallas import tpu as pltpu
from jax.experimental.pallas import tpu_sc as plsc

_ALPHA = 0.5
_MASK_VAL = -100.0
_IT = 2048


def _sc_gather(etab, rtab, eidx, ridx):
    b = eidx.shape[0]
    hdim = etab.shape[1]
    info = plsc.get_sparse_core_info()
    nw = info.num_cores * info.num_subcores
    bw = b // nw
    mesh = plsc.VectorSubcoreMesh(core_axis_name="c", subcore_axis_name="s")

    def body(etab_hbm, rtab_hbm, eidx_hbm, ridx_hbm, h_out, r_out,
             eidx_v, ridx_v, hrows, rrows, sem1, sem2):
        wid = lax.axis_index("s") * info.num_cores + lax.axis_index("c")
        base = wid * bw
        pltpu.sync_copy(eidx_hbm.at[pl.ds(base, bw)], eidx_v)
        pltpu.sync_copy(ridx_hbm.at[pl.ds(base, bw)], ridx_v)
        cp1 = pltpu.async_copy(etab_hbm.at[eidx_v], hrows, sem1)
        cp2 = pltpu.async_copy(rtab_hbm.at[ridx_v], rrows, sem2)
        cp1.wait()
        cp2.wait()
        pltpu.sync_copy(hrows, h_out.at[pl.ds(base, bw)])
        pltpu.sync_copy(rrows, r_out.at[pl.ds(base, bw)])

    call = pl.kernel(
        body,
        mesh=mesh,
        out_type=[
            jax.ShapeDtypeStruct((b, hdim), jnp.float32),
            jax.ShapeDtypeStruct((b, hdim), jnp.float32),
        ],
        scratch_types=[
            pltpu.VMEM((bw,), jnp.int32),
            pltpu.VMEM((bw,), jnp.int32),
            pltpu.VMEM((bw, hdim), jnp.float32),
            pltpu.VMEM((bw, hdim), jnp.float32),
            pltpu.SemaphoreType.DMA,
            pltpu.SemaphoreType.DMA,
        ],
    )
    return call(etab, rtab, eidx, ridx)


def _mt_body(h_ref, r_ref, mt_ref):
    mt_ref[:] = jnp.concatenate([h_ref[:], r_ref[:]],
                                axis=1).astype(jnp.bfloat16)


def _proj(mt_hr, step, wfull_ref, t0_ref, b_ref, h):
    w_hr = wfull_ref[:, : 2 * h].astype(jnp.bfloat16)
    logits = jax.lax.dot_general(mt_hr, w_hr, (((1,), (1,)), ((), ())),
                                 preferred_element_type=jnp.float32)
    v = jax.lax.dot_general(t0_ref[:], wfull_ref[:, 2 * h:],
                            (((1,), (1,)), ((), ())),
                            preferred_element_type=jnp.float32)
    return logits + step * v + b_ref[:]


def _stats_body(mt_ref, step_ref, t0_ref, wg_ref, ws_ref, bg_ref, bs_ref,
                cv_ref, mg_ref, sg_ref, sc_ref, t_ref, *, i_dim, it, h):
    i = pl.program_id(0)
    b = mt_ref.shape[0]

    @pl.when(i == 0)
    def _init():
        mg_ref[:] = jnp.full((b, 1), -jnp.inf, jnp.float32)
        sg_ref[:] = jnp.zeros((b, 1), jnp.float32)
        sc_ref[:] = jnp.zeros((b, 1), jnp.float32)

    col = jax.lax.broadcasted_iota(jnp.int32, (1, it), 1) + i * it
    valid = col < i_dim

    mt = mt_ref[:]
    step = step_ref[:]
    g = _proj(mt, step, wg_ref, t0_ref, bg_ref, h)
    gv = jnp.where(valid, g, -jnp.inf)
    tile_max = jnp.max(gv, axis=1, keepdims=True)
    m_old = mg_ref[:]
    m_new = jnp.maximum(m_old, tile_max)
    e_g = jnp.where(valid, jnp.exp(g - m_new), 0.0)
    sg_ref[:] = sg_ref[:] * jnp.exp(m_old - m_new) + jnp.sum(
        e_g, axis=1, keepdims=True)
    mg_ref[:] = m_new

    s = _proj(mt, step, ws_ref, t0_ref, bs_ref, h)
    q = jnp.tanh(s)
    madd = jnp.where(cv_ref[:] <= 0, _MASK_VAL, 0.0)
    t = jnp.exp(q + madd - 1.0)
    sc_ref[:] = sc_ref[:] + jnp.sum(jnp.where(valid, t, 0.0), axis=1,
                                    keepdims=True)
    t_ref[:] = t.astype(jnp.bfloat16)


def _out_body(mt_ref, step_ref, t0_ref, wg_ref, bg_ref, mg_ref, sg_ref,
              sc_ref, t_ref, out_ref, *, h):
    g = _proj(mt_ref[:], step_ref[:], wg_ref, t0_ref, bg_ref, h)
    score_g = jnp.exp(g - mg_ref[:]) * (1.0 / sg_ref[:])
    inv_sc = 1.0 / jnp.maximum(sc_ref[:], 1e-30)
    score_c = t_ref[:].astype(jnp.float32) * inv_sc
    base = score_c * _ALPHA + score_g * (1.0 - _ALPHA)
    out_ref[:] = jnp.log(jnp.maximum(base, 1e-12))


def _link_prediction(quadruple, copy_vocabulary, ent_init_embeds, w_relation,
                     tim_init_embeds, W_g, b_g, W_s, b_s, *, interpret=False):
    b = quadruple.shape[0]
    i_dim, kdim = W_g.shape
    h = ent_init_embeds.shape[1]
    num_times = 365
    it = min(_IT, i_dim)
    ni = (i_dim + it - 1) // it

    eidx = quadruple[:, 0]
    ridx = quadruple[:, 1]
    step = (jnp.clip(quadruple[:, 3:4], 0, num_times - 1) + 1).astype(
        jnp.float32)
    h_rows, r_rows = _sc_gather(ent_init_embeds, w_relation, eidx, ridx)

    mt_hr = pl.pallas_call(
        _mt_body,
        out_shape=jax.ShapeDtypeStruct((b, 2 * h), jnp.bfloat16),
        interpret=interpret,
    )(h_rows, r_rows)

    bg2 = b_g.reshape(1, i_dim)
    bs2 = b_s.reshape(1, i_dim)
    stats_call = pl.pallas_call(
        functools.partial(_stats_body, i_dim=i_dim, it=it, h=h),
        grid=(ni,),
        in_specs=[
            pl.BlockSpec((b, 2 * h), lambda i: (0, 0)),
            pl.BlockSpec((b, 1), lambda i: (0, 0)),
            pl.BlockSpec((1, h), lambda i: (0, 0)),
            pl.BlockSpec((it, kdim), lambda i: (i, 0)),
            pl.BlockSpec((it, kdim), lambda i: (i, 0)),
            pl.BlockSpec((1, it), lambda i: (0, i)),
            pl.BlockSpec((1, it), lambda i: (0, i)),
            pl.BlockSpec((b, it), lambda i: (0, i)),
        ],
        out_specs=[
            pl.BlockSpec((b, 1), lambda i: (0, 0)),
            pl.BlockSpec((b, 1), lambda i: (0, 0)),
            pl.BlockSpec((b, 1), lambda i: (0, 0)),
            pl.BlockSpec((b, it), lambda i: (0, i)),
        ],
        out_shape=[
            jax.ShapeDtypeStruct((b, 1), jnp.float32),
            jax.ShapeDtypeStruct((b, 1), jnp.float32),
            jax.ShapeDtypeStruct((b, 1), jnp.float32),
            jax.ShapeDtypeStruct((b, i_dim), jnp.bfloat16),
        ],
        compiler_params=pltpu.CompilerParams(
            dimension_semantics=("arbitrary",)),
        interpret=interpret,
    )
    mg, sg, sc, t = stats_call(mt_hr, step, tim_init_embeds, W_g, W_s, bg2,
                               bs2, copy_vocabulary)

    out_call = pl.pallas_call(
        functools.partial(_out_body, h=h),
        grid=(ni,),
        in_specs=[
            pl.BlockSpec((b, 2 * h), lambda i: (0, 0)),
            pl.BlockSpec((b, 1), lambda i: (0, 0)),
            pl.BlockSpec((1, h), lambda i: (0, 0)),
            pl.BlockSpec((it, kdim), lambda i: (i, 0)),
            pl.BlockSpec((1, it), lambda i: (0, i)),
            pl.BlockSpec((b, 1), lambda i: (0, 0)),
            pl.BlockSpec((b, 1), lambda i: (0, 0)),
            pl.BlockSpec((b, 1), lambda i: (0, 0)),
            pl.BlockSpec((b, it), lambda i: (0, i)),
        ],
        out_specs=pl.BlockSpec((b, it), lambda i: (0, i)),
        out_shape=jax.ShapeDtypeStruct((b, i_dim), jnp.float32),
        compiler_params=pltpu.CompilerParams(
            dimension_semantics=("arbitrary",)),
        interpret=interpret,
    )
    return out_call(mt_hr, step, tim_init_embeds, W_g, bg2, mg, sg, sc, t)


def kernel(quadruple, copy_vocabulary, ent_init_embeds, w_relation,
           tim_init_embeds, W_g, b_g, W_s, b_s):
    return _link_prediction(quadruple, copy_vocabulary, ent_init_embeds,
                            w_relation, tim_init_embeds, W_g, b_g, W_s, b_s)
